# baseline (speedup 1.0000x reference)
import functools

import jax
import jax.numpy as jnp
from jax import lax
from jax.experimental import pallas as pl
from jax.experimental.pallas import tpu as pltpu
from jax.experimental.pallas import tpu_sc as plsc

_CHUNK = 32
_NBUF = 4


@functools.lru_cache(maxsize=None)
def _make_gather(num_indices: int, dim: int, dtype):
    info = plsc.get_sparse_core_info()
    nw = info.num_cores * info.num_subcores
    per_w = num_indices // nw
    nchunk = per_w // _CHUNK
    look = _NBUF // 2
    assert num_indices % (nw * _CHUNK) == 0
    assert nchunk % _NBUF == 0 and nchunk >= 2 * _NBUF

    mesh = plsc.VectorSubcoreMesh(
        core_axis_name="core", subcore_axis_name="subcore"
    )

    @functools.partial(
        pl.kernel,
        out_type=jax.ShapeDtypeStruct((num_indices, dim), dtype),
        mesh=mesh,
        scratch_types=[
            pltpu.VMEM((per_w // _CHUNK, _CHUNK), jnp.int32),
        ]
        + [pltpu.VMEM((_CHUNK, dim), dtype) for _ in range(_NBUF)]
        + [pltpu.SemaphoreType.DMA for _ in range(2 * _NBUF)],
    )
    def gather_kernel(table_hbm, ids_hbm, out_hbm, idx_v, *rest):
        bufs = rest[:_NBUF]
        sg = rest[_NBUF:2 * _NBUF]
        ss = rest[2 * _NBUF:3 * _NBUF]

        wid = (lax.axis_index("subcore") * info.num_cores
               + lax.axis_index("core"))
        base = wid * per_w
        pltpu.sync_copy(ids_hbm.at[pl.ds(wid * nchunk, nchunk)], idx_v)

        def issue_gather(cc, b):
            pltpu.async_copy(
                table_hbm.at[idx_v.at[cc]],
                bufs[b], sg[b])

        def wait_gather(b):
            pltpu.make_async_copy(
                table_hbm.at[pl.ds(0, _CHUNK)], bufs[b], sg[b]).wait()

        def issue_store(cc, b):
            pltpu.async_copy(
                bufs[b], out_hbm.at[pl.ds(base + cc * _CHUNK, _CHUNK)],
                ss[b])

        def wait_store(b):
            pltpu.make_async_copy(
                bufs[b], out_hbm.at[pl.ds(0, _CHUNK)], ss[b]).wait()

        def visit(cc, b, prefetch=True, free=True):
            bn = (b + look) % _NBUF
            if free:
                wait_store(bn)
            if prefetch:
                issue_gather(cc + look, bn)
            wait_gather(b)
            issue_store(cc, b)

        for b in range(look):
            issue_gather(b, b)
        for b in range(_NBUF):
            visit(b, b, free=(b >= look))

        @pl.loop(_NBUF, nchunk - _NBUF, step=_NBUF)
        def _(c):
            for b in range(_NBUF):
                visit(c + b, b)

        for b in range(_NBUF):
            visit(nchunk - _NBUF + b, b, prefetch=(b < look),
                  free=(b < look))
        for b in range(_NBUF):
            wait_store(b)

    return gather_kernel


def kernel(token_ids, embedding_table):
    b, s = token_ids.shape
    _, d = embedding_table.shape
    n = b * s
    ids = token_ids.reshape(n // _CHUNK, _CHUNK).astype(jnp.int32)
    out = _make_gather(n, d, embedding_table.dtype)(embedding_table, ids)
    return out.reshape(b, s, d)
